# Initial kernel scaffold; baseline (speedup 1.0000x reference)
import jax
import jax.numpy as jnp
from jax.experimental import pallas as pl


def kernel(coords, features, W1a, b1a, W2a, b2a, W3a, b3a, W1b, b1b, W2b, b2b, W3b, b3b):
    raise NotImplementedError("write your pallas kernel here")


# speedup vs baseline: 1.6000x; 1.6000x over previous
import functools

import jax
import jax.numpy as jnp
from jax import lax
from jax.experimental import pallas as pl
from jax.experimental.pallas import tpu as pltpu

_B, _N = 8, 8192
_M1, _K1 = 512, 32
_M2, _K2 = 128, 16


def _fps_kernel(x_ref, y_ref, z_ref, ox_ref, oy_ref, oz_ref):
    x = x_ref[...]
    y = y_ref[...]
    z = z_ref[...]
    lane_n = lax.broadcasted_iota(jnp.int32, (_B, _N), 1)
    lane_m = lax.broadcasted_iota(jnp.int32, (_B, _M1), 1)

    cx0 = x[:, 0:1]
    cy0 = y[:, 0:1]
    cz0 = z[:, 0:1]
    ox0 = jnp.where(lane_m == 0, cx0, 0.0)
    oy0 = jnp.where(lane_m == 0, cy0, 0.0)
    oz0 = jnp.where(lane_m == 0, cz0, 0.0)
    dmin0 = jnp.full((_B, _N), jnp.inf, dtype=jnp.float32)

    def body(i, carry):
        cx, cy, cz, ox, oy, oz, dmin = carry
        dx = x - cx
        dy = y - cy
        dz = z - cz
        d = dx * dx + dy * dy + dz * dz
        dmin = jnp.minimum(dmin, d)
        m = jnp.max(dmin, axis=1, keepdims=True)
        cand = jnp.where(dmin == m, lane_n, _N)
        nxt = jnp.min(cand, axis=1, keepdims=True)
        sel = lane_n == nxt
        ncx = jnp.sum(jnp.where(sel, x, 0.0), axis=1, keepdims=True)
        ncy = jnp.sum(jnp.where(sel, y, 0.0), axis=1, keepdims=True)
        ncz = jnp.sum(jnp.where(sel, z, 0.0), axis=1, keepdims=True)
        hit = lane_m == i
        ox = jnp.where(hit, ncx, ox)
        oy = jnp.where(hit, ncy, oy)
        oz = jnp.where(hit, ncz, oz)
        return ncx, ncy, ncz, ox, oy, oz, dmin

    _, _, _, ox, oy, oz, _ = lax.fori_loop(
        1, _M1, body, (cx0, cy0, cz0, ox0, oy0, oz0, dmin0))
    ox_ref[...] = ox
    oy_ref[...] = oy
    oz_ref[...] = oz


def _fps(ct):
    return pl.pallas_call(
        _fps_kernel,
        out_shape=[jax.ShapeDtypeStruct((_B, _M1), jnp.float32)] * 3,
    )(ct[0], ct[1], ct[2])


def _enc_kernel(p_ref, w1_ref, b1_ref, w2_ref, b2_ref, w3_ref, b3_ref, o_ref,
                *, group):
    h = jnp.dot(p_ref[...], w1_ref[...], preferred_element_type=jnp.float32)
    h = jnp.maximum(h + b1_ref[...], 0.0)
    h = jnp.dot(h, w2_ref[...], preferred_element_type=jnp.float32)
    h = jnp.maximum(h + b2_ref[...], 0.0)
    h = jnp.dot(h, w3_ref[...], preferred_element_type=jnp.float32)
    h = jnp.maximum(h + b3_ref[...], 0.0)
    r, c = h.shape
    o_ref[...] = jnp.max(h.reshape(r // group, group, c), axis=1)


def _encode(p, w1, b1, w2, b2, w3, b3, group, tile_rows):
    rows, kdim = p.shape
    grid = rows // tile_rows
    otile = tile_rows // group
    c1, c2, c3 = w1.shape[1], w2.shape[1], w3.shape[1]
    return pl.pallas_call(
        functools.partial(_enc_kernel, group=group),
        grid=(grid,),
        in_specs=[
            pl.BlockSpec((tile_rows, kdim), lambda i: (i, 0)),
            pl.BlockSpec((kdim, c1), lambda i: (0, 0)),
            pl.BlockSpec((1, c1), lambda i: (0, 0)),
            pl.BlockSpec((c1, c2), lambda i: (0, 0)),
            pl.BlockSpec((1, c2), lambda i: (0, 0)),
            pl.BlockSpec((c2, c3), lambda i: (0, 0)),
            pl.BlockSpec((1, c3), lambda i: (0, 0)),
        ],
        out_specs=pl.BlockSpec((otile, c3), lambda i: (i, 0)),
        out_shape=jax.ShapeDtypeStruct((rows // group, c3), jnp.float32),
    )(p, w1, b1, w2, b2, w3, b3)


def kernel(coords, features, W1a, b1a, W2a, b2a, W3a, b3a,
           W1b, b1b, W2b, b2b, W3b, b3b):
    ct = jnp.transpose(coords, (2, 0, 1))
    ox, oy, oz = _fps(ct)
    centers1 = jnp.stack([ox, oy, oz], axis=-1)

    bar3 = jnp.arange(_B)[:, None, None]

    cc = jnp.sum(centers1 ** 2, axis=-1)
    xx = jnp.sum(coords ** 2, axis=-1)
    dists = (cc[:, :, None] + xx[:, None, :]
             - 2.0 * jnp.einsum('bmd,bnd->bmn', centers1, coords))
    _, kidx = lax.top_k(-dists, _K1)
    gc = coords[bar3, kidx]
    gf = features[bar3, kidx]
    rel = gc - centers1[:, :, None, :]
    p1 = jnp.concatenate([rel, gf], axis=-1)
    p1 = jnp.pad(p1, ((0, 0), (0, 0), (0, 0), (0, 2)))
    w1a = jnp.pad(W1a, ((0, 2), (0, 0)))
    x1 = _encode(p1.reshape(_B * _M1 * _K1, 8), w1a, b1a.reshape(1, -1),
                 W2a, b2a.reshape(1, -1), W3a, b3a.reshape(1, -1),
                 _K1, 2048).reshape(_B, _M1, 128)

    centers2 = centers1[:, :_M2]
    cc2 = cc[:, :_M2]
    d2 = (cc2[:, :, None] + cc[:, None, :]
          - 2.0 * jnp.einsum('bmd,bnd->bmn', centers2, centers1))
    _, kidx2 = lax.top_k(-d2, _K2)
    gc2 = centers1[bar3, kidx2]
    gf2 = x1[bar3, kidx2]
    rel2 = gc2 - centers2[:, :, None, :]
    p2 = jnp.concatenate([rel2, gf2], axis=-1)
    p2 = jnp.pad(p2, ((0, 0), (0, 0), (0, 0), (0, 5)))
    w1b = jnp.pad(W1b, ((0, 5), (0, 0)))
    x2 = _encode(p2.reshape(_B * _M2 * _K2, 136), w1b, b1b.reshape(1, -1),
                 W2b, b2b.reshape(1, -1), W3b, b3b.reshape(1, -1),
                 _K2, 2048).reshape(_B, _M2, 256)

    return (centers1, x1, centers2, x2)


# speedup vs baseline: 12.5174x; 7.8233x over previous
import functools

import jax
import jax.numpy as jnp
from jax import lax
from jax.experimental import pallas as pl
from jax.experimental.pallas import tpu as pltpu
from jax.experimental.pallas import tpu_sc as plsc

_B, _N = 8, 8192
_M1, _K1 = 512, 32
_M2, _K2 = 128, 16
_NW = 32


def _fps_kernel(x_ref, y_ref, z_ref, ox_ref, oy_ref, oz_ref):
    x = x_ref[...]
    y = y_ref[...]
    z = z_ref[...]
    lane_n = lax.broadcasted_iota(jnp.int32, (_B, _N), 1)
    lane_m = lax.broadcasted_iota(jnp.int32, (_B, _M1), 1)

    cx0 = x[:, 0:1]
    cy0 = y[:, 0:1]
    cz0 = z[:, 0:1]
    ox0 = jnp.where(lane_m == 0, cx0, 0.0)
    oy0 = jnp.where(lane_m == 0, cy0, 0.0)
    oz0 = jnp.where(lane_m == 0, cz0, 0.0)
    dmin0 = jnp.full((_B, _N), jnp.inf, dtype=jnp.float32)

    def body(i, carry):
        cx, cy, cz, ox, oy, oz, dmin = carry
        dx = x - cx
        dy = y - cy
        dz = z - cz
        d = dx * dx + dy * dy + dz * dz
        dmin = jnp.minimum(dmin, d)
        m = jnp.max(dmin, axis=1, keepdims=True)
        cand = jnp.where(dmin == m, lane_n, _N)
        nxt = jnp.min(cand, axis=1, keepdims=True)
        sel = lane_n == nxt
        ncx = jnp.sum(jnp.where(sel, x, 0.0), axis=1, keepdims=True)
        ncy = jnp.sum(jnp.where(sel, y, 0.0), axis=1, keepdims=True)
        ncz = jnp.sum(jnp.where(sel, z, 0.0), axis=1, keepdims=True)
        hit = lane_m == i
        ox = jnp.where(hit, ncx, ox)
        oy = jnp.where(hit, ncy, oy)
        oz = jnp.where(hit, ncz, oz)
        return ncx, ncy, ncz, ox, oy, oz, dmin

    _, _, _, ox, oy, oz, _ = lax.fori_loop(
        1, _M1, body, (cx0, cy0, cz0, ox0, oy0, oz0, dmin0))
    ox_ref[...] = ox
    oy_ref[...] = oy
    oz_ref[...] = oz


def _fps(ct):
    return pl.pallas_call(
        _fps_kernel,
        out_shape=[jax.ShapeDtypeStruct((_B, _M1), jnp.float32)] * 3,
    )(ct[0], ct[1], ct[2])


def _knn_kernel(ct_ref, cen_ref, kidx_ref, d_ref, *, m, n, k):
    ct = ct_ref[0]
    cen = cen_ref[0]
    xx = jnp.sum(ct * ct, axis=0, keepdims=True)
    cc = jnp.sum(cen * cen, axis=1, keepdims=True)
    e = jnp.dot(cen, ct, preferred_element_type=jnp.float32)
    d_ref[...] = cc + xx - 2.0 * e
    lane_n = lax.broadcasted_iota(jnp.int32, (m, n), 1)
    lane_k = lax.broadcasted_iota(jnp.int32, (m, k), 1)

    def step(j, kidx):
        dcur = d_ref[...]
        mn = jnp.min(dcur, axis=1, keepdims=True)
        cand = jnp.where(dcur == mn, lane_n, n)
        nxt = jnp.min(cand, axis=1, keepdims=True)
        d_ref[...] = jnp.where(lane_n == nxt, jnp.inf, dcur)
        return jnp.where(lane_k == j, nxt, kidx)

    kidx_ref[0] = lax.fori_loop(0, k, step, jnp.zeros((m, k), jnp.int32))


def _knn(ct8, cen, m, n, k):
    return pl.pallas_call(
        functools.partial(_knn_kernel, m=m, n=n, k=k),
        grid=(_B,),
        in_specs=[
            pl.BlockSpec((1, 8, n), lambda b: (b, 0, 0)),
            pl.BlockSpec((1, m, 8), lambda b: (b, 0, 0)),
        ],
        out_specs=pl.BlockSpec((1, m, k), lambda b: (b, 0, 0)),
        out_shape=jax.ShapeDtypeStruct((_B, m, k), jnp.int32),
        scratch_shapes=[pltpu.VMEM((m, n), jnp.float32)],
    )(ct8, cen)


def _sc_gather(table, idx):
    ni = idx.shape[0]
    d = table.shape[1]
    bpw = ni // _NW
    mesh = plsc.VectorSubcoreMesh(core_axis_name="c", subcore_axis_name="s")

    @functools.partial(
        pl.kernel,
        out_type=jax.ShapeDtypeStruct((ni, d), jnp.float32),
        mesh=mesh,
        scratch_types=[
            pltpu.VMEM((bpw,), jnp.int32),
            pltpu.VMEM((bpw, d), jnp.float32),
            pltpu.SemaphoreType.DMA,
        ],
        compiler_params=pltpu.CompilerParams(use_tc_tiling_on_sc=False),
    )
    def gk(table_hbm, idx_hbm, out_hbm, idx_v, rows_v, sem):
        wid = lax.axis_index("s") * 2 + lax.axis_index("c")
        base = wid * bpw
        pltpu.sync_copy(idx_hbm.at[pl.ds(base, bpw)], idx_v)
        pltpu.async_copy(table_hbm.at[idx_v], rows_v, sem).wait()
        pltpu.sync_copy(rows_v, out_hbm.at[pl.ds(base, bpw)])

    return gk(table, idx)


def _enc_kernel(p_ref, c_ref, w1_ref, b1_ref, w2_ref, b2_ref, w3_ref, b3_ref,
                o_ref, *, group):
    g = p_ref[...]
    c = c_ref[...]
    r, kd = g.shape
    cb = jnp.broadcast_to(c[:, None, :], (r // group, group, kd)).reshape(r, kd)
    h = g - cb
    h = jnp.dot(h, w1_ref[...], preferred_element_type=jnp.float32)
    h = jnp.maximum(h + b1_ref[...], 0.0)
    h = jnp.dot(h, w2_ref[...], preferred_element_type=jnp.float32)
    h = jnp.maximum(h + b2_ref[...], 0.0)
    h = jnp.dot(h, w3_ref[...], preferred_element_type=jnp.float32)
    h = jnp.maximum(h + b3_ref[...], 0.0)
    o_ref[...] = jnp.max(h.reshape(r // group, group, -1), axis=1)


def _encode(p, cen, w1, b1, w2, b2, w3, b3, group, tile_rows):
    rows, kdim = p.shape
    grid = rows // tile_rows
    otile = tile_rows // group
    c1, c2, c3 = w1.shape[1], w2.shape[1], w3.shape[1]
    return pl.pallas_call(
        functools.partial(_enc_kernel, group=group),
        grid=(grid,),
        in_specs=[
            pl.BlockSpec((tile_rows, kdim), lambda i: (i, 0)),
            pl.BlockSpec((otile, kdim), lambda i: (i, 0)),
            pl.BlockSpec((kdim, c1), lambda i: (0, 0)),
            pl.BlockSpec((1, c1), lambda i: (0, 0)),
            pl.BlockSpec((c1, c2), lambda i: (0, 0)),
            pl.BlockSpec((1, c2), lambda i: (0, 0)),
            pl.BlockSpec((c2, c3), lambda i: (0, 0)),
            pl.BlockSpec((1, c3), lambda i: (0, 0)),
        ],
        out_specs=pl.BlockSpec((otile, c3), lambda i: (i, 0)),
        out_shape=jax.ShapeDtypeStruct((rows // group, c3), jnp.float32),
    )(p, cen, w1, b1, w2, b2, w3, b3)


def kernel(coords, features, W1a, b1a, W2a, b2a, W3a, b3a,
           W1b, b1b, W2b, b2b, W3b, b3b):
    ct = jnp.transpose(coords, (2, 0, 1))
    ox, oy, oz = _fps(ct)
    centers1 = jnp.stack([ox, oy, oz], axis=-1)
    zm = jnp.zeros((_B, _M1), jnp.float32)
    cen_pad = jnp.stack([ox, oy, oz, zm, zm, zm, zm, zm], axis=-1)
    ct8 = jnp.concatenate(
        [ct, jnp.zeros((5, _B, _N), jnp.float32)], axis=0
    ).transpose(1, 0, 2)

    kidx = _knn(ct8, cen_pad, _M1, _N, _K1)
    gidx = kidx + (jnp.arange(_B, dtype=jnp.int32) * _N)[:, None, None]
    table1 = jnp.concatenate([coords, features], axis=-1)
    table1 = jnp.pad(table1, ((0, 0), (0, 0), (0, 10))).reshape(_B * _N, 16)
    g1 = _sc_gather(table1, gidx.reshape(-1))
    cen16 = jnp.pad(centers1, ((0, 0), (0, 0), (0, 13))).reshape(_B * _M1, 16)
    w1a = jnp.pad(W1a, ((0, 10), (0, 0)))
    x1 = _encode(g1, cen16, w1a, b1a.reshape(1, -1), W2a, b2a.reshape(1, -1),
                 W3a, b3a.reshape(1, -1), _K1, 2048)
    x1r = x1.reshape(_B, _M1, 128)

    ct2 = jnp.stack([ox, oy, oz, zm, zm, zm, zm, zm], axis=1)
    kidx2 = _knn(ct2, cen_pad[:, :_M2], _M2, _M1, _K2)
    gidx2 = kidx2 + (jnp.arange(_B, dtype=jnp.int32) * _M1)[:, None, None]
    table2 = jnp.concatenate(
        [centers1, x1r, jnp.zeros((_B, _M1, 13), jnp.float32)], axis=-1
    ).reshape(_B * _M1, 144)
    g2 = _sc_gather(table2, gidx2.reshape(-1))
    cen144 = jnp.pad(centers1[:, :_M2],
                     ((0, 0), (0, 0), (0, 141))).reshape(_B * _M2, 144)
    w1b = jnp.pad(W1b, ((0, 13), (0, 0)))
    x2 = _encode(g2, cen144, w1b, b1b.reshape(1, -1), W2b, b2b.reshape(1, -1),
                 W3b, b3b.reshape(1, -1), _K2, 2048)
    x2r = x2.reshape(_B, _M2, 256)

    return (centers1, x1r, centers1[:, :_M2], x2r)
